# baseline (speedup 1.0000x reference)
import functools

import jax
import jax.numpy as jnp
from jax import lax
from jax.experimental import pallas as pl
from jax.experimental.pallas import tpu as pltpu
from jax.experimental.pallas import tpu_sc as plsc

N = 10000
D = 128
HID = 64
HEADS = 8
E = 320000

NPAD = 10240
RPT = NPAD // 16
E_TOT = E + N
E_PAD = 344064
EH = E_PAD // 2
CE1 = E_PAD // 32
K1 = 512
CE2 = E_PAD // 16
K2 = 64
NG = 4
EXTRA = 3 * K2

_mesh = plsc.VectorSubcoreMesh(core_axis_name="c", subcore_axis_name="s")


_R1 = 400


def _tc_prologue_body(x_ref, w_ref, avs_ref, avd_ref, xlg_ref, als_ref, ald_ref):
    xl = jnp.dot(x_ref[...], w_ref[...], preferred_element_type=jnp.float32,
                 precision=lax.Precision.HIGHEST)
    ch = lax.broadcasted_iota(jnp.int32, (HEADS * HID, HEADS), 0)
    hh = lax.broadcasted_iota(jnp.int32, (HEADS * HID, HEADS), 1)
    sel = (ch // HID == hh).astype(jnp.float32)
    als_ref[...] = jnp.dot(xl * avs_ref[...], sel,
                           preferred_element_type=jnp.float32,
                           precision=lax.Precision.HIGHEST)
    ald_ref[...] = jnp.dot(xl * avd_ref[...], sel,
                           preferred_element_type=jnp.float32,
                           precision=lax.Precision.HIGHEST)
    for g in range(NG):
        xlg_ref[g] = xl[:, g * 128:(g + 1) * 128]


def _tc_prologue(x, W1, avs, avd):
    return pl.pallas_call(
        _tc_prologue_body,
        grid=(N // _R1,),
        in_specs=[
            pl.BlockSpec((_R1, D), lambda i: (i, 0)),
            pl.BlockSpec((D, HEADS * HID), lambda i: (0, 0)),
            pl.BlockSpec((1, HEADS * HID), lambda i: (0, 0)),
            pl.BlockSpec((1, HEADS * HID), lambda i: (0, 0)),
        ],
        out_specs=[
            pl.BlockSpec((NG, _R1, 128), lambda i: (0, i, 0)),
            pl.BlockSpec((_R1, HEADS), lambda i: (i, 0)),
            pl.BlockSpec((_R1, HEADS), lambda i: (i, 0)),
        ],
        out_shape=[
            jax.ShapeDtypeStruct((NG, N, 128), jnp.float32),
            jax.ShapeDtypeStruct((N, HEADS), jnp.float32),
            jax.ShapeDtypeStruct((N, HEADS), jnp.float32),
        ],
    )(x, W1, avs, avd)


NCH1 = E_PAD // K1
NI1 = NCH1 // 32
W_LEN = E_PAD * 8 + 2 * K1 * 8


def _sc_pass1_body(idxs_hbm, idxd_hbm, als_hbm, ald_hbm, zeros_hbm,
                   w_hbm, denom_hbm,
                   is0, is1, is2, id0, id1, id2, s0, s1, d0, d1, w0, w1,
                   bounce_v, denom_sh,
                   semL0, semL1, semL2, semG0, semG1, semW0, semW1):
    c = lax.axis_index("c")
    s = lax.axis_index("s")
    wid = s * 2 + c
    idxs = (is0, is1, is2)
    idxd = (id0, id1, id2)
    s_v = (s0, s1)
    d_v = (d0, d1)
    w_v = (w0, w1)
    semL = (semL0, semL1, semL2)
    semG = (semG0, semG1)
    semW = (semW0, semW1)

    z0 = s * (RPT * 8)
    pltpu.sync_copy(zeros_hbm.at[pl.ds(z0, RPT * 8)], bounce_v)
    pltpu.sync_copy(bounce_v, denom_sh.at[pl.ds(z0, RPT * 8)])
    plsc.subcore_barrier()

    def f0_of(i):
        t = jnp.minimum(i * 32 + wid, NCH1 - 1)
        return t * (K1 * 8)

    def fire_lin(i, l):
        f0 = f0_of(i)
        pltpu.async_copy(idxs_hbm.at[pl.ds(f0, K1 * 8)], idxs[l], semL[l])
        pltpu.async_copy(idxd_hbm.at[pl.ds(f0, K1 * 8)], idxd[l], semL[l])

    def wait_lin(l):
        pltpu.make_async_copy(idxs_hbm.at[pl.ds(0, K1 * 8)], idxs[l],
                              semL[l]).wait()
        pltpu.make_async_copy(idxd_hbm.at[pl.ds(0, K1 * 8)], idxd[l],
                              semL[l]).wait()

    def fire_gather(l, o):
        pltpu.async_copy(als_hbm.at[idxs[l]], s_v[o], semG[o])
        pltpu.async_copy(ald_hbm.at[idxd[l]], d_v[o], semG[o])

    def wait_gather(b):
        pltpu.make_async_copy(als_hbm.at[idxs[0]], s_v[b], semG[b]).wait()
        pltpu.make_async_copy(ald_hbm.at[idxd[0]], d_v[b], semG[b]).wait()

    fire_lin(0, 0)
    fire_lin(1, 1)
    fire_lin(2, 2)
    wait_lin(0)
    fire_gather(0, 0)
    for b in range(2):
        pltpu.async_copy(w_v[b],
                         w_hbm.at[pl.ds(E_PAD * 8 + b * (K1 * 8), K1 * 8)],
                         semW[b])

    def body(i, bi):
        b = bi % 2
        l = bi % 3
        l1 = (bi + 1) % 3
        o = 1 - b
        wait_lin(l1)
        fire_gather(l1, o)
        wait_gather(b)
        pltpu.make_async_copy(w_v[b], w_hbm.at[pl.ds(0, K1 * 8)],
                              semW[b]).wait()

        def vec(j, carry2, b=b):
            e = s_v[b][pl.ds(j * 16, 16)] + d_v[b][pl.ds(j * 16, 16)]
            e = jnp.maximum(e, 0.2 * e)
            w_v[b][pl.ds(j * 16, 16)] = jnp.exp(e)
            return carry2

        lax.fori_loop(0, (K1 * 8) // 16, vec, 0)
        pltpu.async_copy(w_v[b], w_hbm.at[pl.ds(f0_of(i), K1 * 8)],
                         semW[b])
        pltpu.sync_copy(w_v[b], denom_sh.at[idxd[l]], add=True)
        fire_lin(i + 3, l)

    def outer(oi, carry):
        ibase = oi * 6
        for bi in range(6):
            body(ibase + bi, bi)
        return carry

    lax.fori_loop(0, (NI1 // 6) * 6 // 6, outer, 0)
    for j in range(NI1 % 6):
        body((NI1 // 6) * 6 + j, j)
    wait_lin((NI1 + 1) % 3)
    wait_lin((NI1 + 2) % 3)
    wait_gather(NI1 % 2)
    pltpu.make_async_copy(w_v[0], w_hbm.at[pl.ds(0, K1 * 8)], semW[0]).wait()
    pltpu.make_async_copy(w_v[1], w_hbm.at[pl.ds(0, K1 * 8)], semW[1]).wait()

    plsc.subcore_barrier()
    pltpu.sync_copy(denom_sh.at[pl.ds(z0, RPT * 8)], bounce_v)
    pltpu.sync_copy(bounce_v,
                    denom_hbm.at[pl.ds(c * (NPAD * 8) + z0, RPT * 8)])


_sc_pass1 = functools.partial(
    pl.kernel,
    out_type=[
        jax.ShapeDtypeStruct((W_LEN,), jnp.float32),
        jax.ShapeDtypeStruct((2 * NPAD * 8,), jnp.float32),
    ],
    mesh=_mesh,
    scratch_types=[
        pltpu.VMEM((K1 * 8,), jnp.int32),
        pltpu.VMEM((K1 * 8,), jnp.int32),
        pltpu.VMEM((K1 * 8,), jnp.int32),
        pltpu.VMEM((K1 * 8,), jnp.int32),
        pltpu.VMEM((K1 * 8,), jnp.int32),
        pltpu.VMEM((K1 * 8,), jnp.int32),
        pltpu.VMEM((K1 * 8,), jnp.float32),
        pltpu.VMEM((K1 * 8,), jnp.float32),
        pltpu.VMEM((K1 * 8,), jnp.float32),
        pltpu.VMEM((K1 * 8,), jnp.float32),
        pltpu.VMEM((K1 * 8,), jnp.float32),
        pltpu.VMEM((K1 * 8,), jnp.float32),
        pltpu.VMEM((RPT * 8,), jnp.float32),
        pltpu.VMEM_SHARED((NPAD * 8,), jnp.float32),
        pltpu.SemaphoreType.DMA,
        pltpu.SemaphoreType.DMA,
        pltpu.SemaphoreType.DMA,
        pltpu.SemaphoreType.DMA,
        pltpu.SemaphoreType.DMA,
        pltpu.SemaphoreType.DMA,
        pltpu.SemaphoreType.DMA,
    ],
)(_sc_pass1_body)


_QR = RPT // 16


def _sc_pass2_body(xlg_hbm, src_hbm, dst_hbm, w_hbm, zeros_hbm,
                   accum_hbm,
                   src0, src1, src2, dst0, dst1, dst2, idx0, idx1, idx2,
                   w0, w1, w2, rows0, rows1, msg0, msg1, sdst0, sdst1,
                   bounce_v, accum_sh,
                   semL0, semL1, semL2, semG0, semG1, semS0, semS1):
    c = lax.axis_index("c")
    s = lax.axis_index("s")
    r0 = s * RPT
    e_base = s * CE2
    srcs = (src0, src1, src2)
    dsts = (dst0, dst1, dst2)
    idxs = (idx0, idx1, idx2)
    ws = (w0, w1, w2)
    rows = (rows0, rows1)
    msgs = (msg0, msg1)
    sdsts = (sdst0, sdst1)
    semL = (semL0, semL1, semL2)
    semG = (semG0, semG1)
    semS = (semS0, semS1)
    NCH = CE2 // K2

    def fire_lin(cc, l):
        e0 = e_base + cc * K2
        pltpu.async_copy(src_hbm.at[pl.ds(e0, K2)], srcs[l], semL[l])
        pltpu.async_copy(dst_hbm.at[pl.ds(e0, K2)], dsts[l], semL[l])
        pltpu.async_copy(w_hbm.at[pl.ds(e0 * 8, K2 * 8)], ws[l], semL[l])

    def wait_lin(l):
        pltpu.make_async_copy(src_hbm.at[pl.ds(0, K2)], srcs[l],
                              semL[l]).wait()
        pltpu.make_async_copy(dst_hbm.at[pl.ds(0, K2)], dsts[l],
                              semL[l]).wait()
        pltpu.make_async_copy(w_hbm.at[pl.ds(0, K2 * 8)], ws[l],
                              semL[l]).wait()

    def build_idx(l, g):
        for j in range(K2 // 16):
            idxs[l][pl.ds(j * 16, 16)] = (srcs[l][pl.ds(j * 16, 16)]
                                          + g * N)

    for gi in range(2):
        g = 2 * c + gi
        for q in range(16):
            pltpu.sync_copy(zeros_hbm.at[pl.ds(r0 + q * _QR, _QR)], bounce_v)
            pltpu.sync_copy(bounce_v, accum_sh.at[pl.ds(r0 + q * _QR, _QR)])
        plsc.subcore_barrier()

        for b in range(2):
            def zmsg(k, carry, b=b):
                for j in range(8):
                    msgs[b][k, pl.ds(j * 16, 16)] = jnp.zeros((16,),
                                                              jnp.float32)
                return carry
            lax.fori_loop(0, K2, zmsg, 0)
            for j in range(4):
                sdsts[b][pl.ds(j * 16, 16)] = jnp.zeros((16,), jnp.int32)
            pltpu.async_copy(msgs[b], accum_sh.at[sdsts[b]], semS[b],
                             add=True)

        fire_lin(0, 0)
        fire_lin(1, 1)
        fire_lin(2, 2)
        wait_lin(0)
        build_idx(0, g)
        pltpu.async_copy(xlg_hbm.at[idxs[0]], rows[0], semG[0])

        def outer(oi, carry, g=g):
            cbase = oi * 6
            for bi in range(6):
                b = bi % 2
                l = bi % 3
                l1 = (bi + 1) % 3
                o = 1 - b
                cc = cbase + bi
                wait_lin(l1)
                build_idx(l1, g)
                pltpu.async_copy(xlg_hbm.at[idxs[l1]], rows[o], semG[o])
                pltpu.make_async_copy(xlg_hbm.at[idxs[l]], rows[b],
                                      semG[b]).wait()
                pltpu.make_async_copy(msgs[b], accum_sh.at[sdsts[b]],
                                      semS[b]).wait()

                def pair(kk, carry2, b=b, l=l, g=g):
                    wrow = ws[l][pl.ds(kk * 16, 16)]
                    for half in range(2):
                        k = kk * 2 + half
                        lo = half * 8 + 2 * g
                        wlo = wrow.at[jnp.full((16,), lo, jnp.int32)].get(
                            mode="promise_in_bounds")
                        whi = wrow.at[jnp.full((16,), lo + 1,
                                               jnp.int32)].get(
                            mode="promise_in_bounds")
                        for j in range(8):
                            wv = wlo if j < 4 else whi
                            msgs[b][k, pl.ds(j * 16, 16)] = (
                                rows[b][k, pl.ds(j * 16, 16)] * wv)
                    return carry2

                lax.fori_loop(0, K2 // 2, pair, 0)
                for j in range(4):
                    sdsts[b][pl.ds(j * 16, 16)] = dsts[l][pl.ds(j * 16, 16)]
                pltpu.async_copy(msgs[b], accum_sh.at[sdsts[b]], semS[b],
                                 add=True)
                fire_lin(cc + 3, l)
            return carry

        lax.fori_loop(0, NCH // 6, outer, 0)
        wait_lin((NCH + 1) % 3)
        wait_lin((NCH + 2) % 3)
        pltpu.make_async_copy(xlg_hbm.at[idxs[NCH % 3]], rows[NCH % 2],
                              semG[NCH % 2]).wait()
        pltpu.make_async_copy(msgs[0], accum_sh.at[sdsts[0]], semS[0]).wait()
        pltpu.make_async_copy(msgs[1], accum_sh.at[sdsts[1]], semS[1]).wait()
        plsc.subcore_barrier()
        for q in range(16):
            pltpu.sync_copy(accum_sh.at[pl.ds(r0 + q * _QR, _QR)], bounce_v)
            pltpu.sync_copy(bounce_v,
                            accum_hbm.at[g, pl.ds(r0 + q * _QR, _QR)])
        plsc.subcore_barrier()


_sc_pass2 = functools.partial(
    pl.kernel,
    out_type=[jax.ShapeDtypeStruct((NG, NPAD, 128), jnp.float32)],
    mesh=_mesh,
    scratch_types=[
        pltpu.VMEM((K2,), jnp.int32),
        pltpu.VMEM((K2,), jnp.int32),
        pltpu.VMEM((K2,), jnp.int32),
        pltpu.VMEM((K2,), jnp.int32),
        pltpu.VMEM((K2,), jnp.int32),
        pltpu.VMEM((K2,), jnp.int32),
        pltpu.VMEM((K2,), jnp.int32),
        pltpu.VMEM((K2,), jnp.int32),
        pltpu.VMEM((K2,), jnp.int32),
        pltpu.VMEM((K2 * 8,), jnp.float32),
        pltpu.VMEM((K2 * 8,), jnp.float32),
        pltpu.VMEM((K2 * 8,), jnp.float32),
        pltpu.VMEM((K2, 128), jnp.float32),
        pltpu.VMEM((K2, 128), jnp.float32),
        pltpu.VMEM((K2, 128), jnp.float32),
        pltpu.VMEM((K2, 128), jnp.float32),
        pltpu.VMEM((K2,), jnp.int32),
        pltpu.VMEM((K2,), jnp.int32),
        pltpu.VMEM((_QR, 128), jnp.float32),
        pltpu.VMEM_SHARED((NPAD, 128), jnp.float32),
        pltpu.SemaphoreType.DMA,
        pltpu.SemaphoreType.DMA,
        pltpu.SemaphoreType.DMA,
        pltpu.SemaphoreType.DMA,
        pltpu.SemaphoreType.DMA,
        pltpu.SemaphoreType.DMA,
        pltpu.SemaphoreType.DMA,
    ],
)(_sc_pass2_body)


def _tc_epi1_body(acc_ref, dn_ref, b1_ref, w2_ref, z_ref):
    dn = dn_ref[0] + dn_ref[1] + 1e-16
    rdn = 1.0 / dn
    ch = lax.broadcasted_iota(jnp.int32, (HEADS, HEADS * HID), 1)
    hh = lax.broadcasted_iota(jnp.int32, (HEADS, HEADS * HID), 0)
    sel = (ch // HID == hh).astype(jnp.float32)
    rdn_full = jnp.dot(rdn, sel, preferred_element_type=jnp.float32,
                       precision=lax.Precision.HIGHEST)
    acc = jnp.concatenate([acc_ref[g] for g in range(NG)], axis=1)
    h = jnp.maximum(acc * rdn_full + b1_ref[...], 0.0)
    z_ref[...] = jnp.dot(h, w2_ref[...], preferred_element_type=jnp.float32,
                         precision=lax.Precision.HIGHEST)


_RE1 = 1280


def _tc_epi1(accum, denom, b1, W2):
    return pl.pallas_call(
        _tc_epi1_body,
        grid=(NPAD // _RE1,),
        in_specs=[
            pl.BlockSpec((NG, _RE1, 128), lambda i: (0, i, 0)),
            pl.BlockSpec((2, _RE1, HEADS), lambda i: (0, i, 0)),
            pl.BlockSpec((1, HEADS * HID), lambda i: (0, 0)),
            pl.BlockSpec((HEADS * HID, 1), lambda i: (0, 0)),
        ],
        out_specs=pl.BlockSpec((_RE1, 1), lambda i: (i, 0)),
        out_shape=jax.ShapeDtypeStruct((NPAD, 1), jnp.float32),
    )(accum, denom, b1, W2)


def _sc_pass3_body(src_hbm, dst_hbm, als_hbm, ald_hbm, z_hbm, zeros_hbm,
                   nd_hbm,
                   src_v, dst_v, s_v, d_v, z_v, w_v, m_v, bounce_v, nd_sh,
                   sem1, sem2, sem3):
    c = lax.axis_index("c")
    s = lax.axis_index("s")
    z0 = s * (2 * RPT)
    pltpu.sync_copy(zeros_hbm.at[pl.ds(z0, 2 * RPT)], bounce_v)
    pltpu.sync_copy(bounce_v, nd_sh.at[pl.ds(z0, 2 * RPT)])
    plsc.subcore_barrier()

    e_base = c * EH + s * CE1

    def chunk(i, carry):
        e0 = e_base + i * K1
        pltpu.sync_copy(src_hbm.at[pl.ds(e0, K1)], src_v)
        pltpu.sync_copy(dst_hbm.at[pl.ds(e0, K1)], dst_v)
        pltpu.async_copy(als_hbm.at[src_v], s_v, sem1).wait()
        pltpu.async_copy(ald_hbm.at[dst_v], d_v, sem2).wait()
        pltpu.async_copy(z_hbm.at[src_v], z_v, sem3).wait()

        def vec(j, carry2):
            e = s_v[pl.ds(j * 16, 16)] + d_v[pl.ds(j * 16, 16)]
            e = jnp.maximum(e, 0.2 * e)
            w = jnp.exp(e)
            w_v[pl.ds(j * 16, 16)] = w
            m_v[pl.ds(j * 16, 16)] = w * z_v[pl.ds(j * 16, 16)]
            return carry2

        lax.fori_loop(0, K1 // 16, vec, 0)
        pltpu.sync_copy(w_v, nd_sh.at[dst_v], add=True)

        def mkidx(j, carry2):
            dst_v[pl.ds(j * 16, 16)] = dst_v[pl.ds(j * 16, 16)] + NPAD
            return carry2

        lax.fori_loop(0, K1 // 16, mkidx, 0)
        pltpu.sync_copy(m_v, nd_sh.at[dst_v], add=True)
        return carry

    lax.fori_loop(0, CE1 // K1, chunk, 0)
    plsc.subcore_barrier()
    pltpu.sync_copy(nd_sh.at[pl.ds(z0, 2 * RPT)], bounce_v)
    pltpu.sync_copy(bounce_v,
                    nd_hbm.at[pl.ds(c * (2 * NPAD) + z0, 2 * RPT)])


_sc_pass3 = functools.partial(
    pl.kernel,
    out_type=[jax.ShapeDtypeStruct((2 * 2 * NPAD,), jnp.float32)],
    mesh=_mesh,
    scratch_types=[
        pltpu.VMEM((K1,), jnp.int32),
        pltpu.VMEM((K1,), jnp.int32),
        pltpu.VMEM((K1,), jnp.float32),
        pltpu.VMEM((K1,), jnp.float32),
        pltpu.VMEM((K1,), jnp.float32),
        pltpu.VMEM((K1,), jnp.float32),
        pltpu.VMEM((K1,), jnp.float32),
        pltpu.VMEM((2 * RPT,), jnp.float32),
        pltpu.VMEM_SHARED((2 * NPAD,), jnp.float32),
        pltpu.SemaphoreType.DMA,
        pltpu.SemaphoreType.DMA,
        pltpu.SemaphoreType.DMA,
    ],
)(_sc_pass3_body)


def _tc_epi2_body(nd_ref, b2_ref, o_ref):
    dn = nd_ref[0, 0] + nd_ref[1, 0]
    nm = nd_ref[0, 1] + nd_ref[1, 1]
    o_ref[...] = nm / (dn + 1e-16) + b2_ref[0, 0]


def _tc_epi2(nd, b2):
    return pl.pallas_call(
        _tc_epi2_body,
        in_specs=[
            pl.BlockSpec((2, 2, NPAD // 64, 64), lambda: (0, 0, 0, 0)),
            pl.BlockSpec(memory_space=pltpu.SMEM),
        ],
        out_specs=pl.BlockSpec((NPAD // 64, 64), lambda: (0, 0)),
        out_shape=jax.ShapeDtypeStruct((NPAD // 64, 64), jnp.float32),
        grid=(),
    )(nd, b2)


def kernel(x, edge_index, W1, a_s1, a_d1, b1, W2, a_s2, a_d2, b2):
    f32 = jnp.float32
    loop = jnp.arange(N, dtype=jnp.int32)
    src = jnp.concatenate([edge_index[0].astype(jnp.int32), loop])
    dst = jnp.concatenate([edge_index[1].astype(jnp.int32), loop])
    npad_e = E_PAD - E_TOT
    pidx = jnp.arange(npad_e, dtype=jnp.int32)
    ztail = jnp.zeros((EXTRA,), jnp.int32)
    src_full = jnp.concatenate([src, pidx % N, ztail])
    dst_full = jnp.concatenate([dst, N + (pidx % 16), ztail])

    h8 = jnp.arange(8, dtype=jnp.int32)
    idx_s8 = (src_full[:E_PAD, None] * 8 + h8[None, :]).reshape(-1)
    idx_d8 = (dst_full[:E_PAD, None] * 8 + h8[None, :]).reshape(-1)

    avs = a_s1.reshape(1, HEADS * HID)
    avd = a_d1.reshape(1, HEADS * HID)
    xlg, als, ald = _tc_prologue(x, W1, avs, avd)
    xlg_flat = xlg.reshape(NG * N, 128)
    als_flat = jnp.pad(als, ((0, NPAD - N), (0, 0))).reshape(-1)
    ald_flat = jnp.pad(ald, ((0, NPAD - N), (0, 0))).reshape(-1)

    zeros8 = jnp.zeros((NPAD * 8,), f32)
    w_flat, denom = _sc_pass1(idx_s8, idx_d8, als_flat, ald_flat, zeros8)

    zeros128 = jnp.zeros((NPAD, 128), f32)
    (accum,) = _sc_pass2(xlg_flat, src_full, dst_full, w_flat, zeros128)

    denom3 = denom.reshape(2, NPAD, HEADS)
    b1r = b1.reshape(1, HEADS * HID)
    z = _tc_epi1(accum, denom3, b1r, W2)

    z_flat = z.reshape(-1)
    as2 = a_s2.reshape(())
    ad2 = a_d2.reshape(())
    als2 = z_flat * as2
    ald2 = z_flat * ad2

    zeros2 = jnp.zeros((2 * NPAD,), f32)
    (nd,) = _sc_pass3(src_full, dst_full, als2, ald2, z_flat, zeros2)

    nd4 = nd.reshape(2, 2, NPAD // 64, 64)
    b2r = b2.reshape(1, 1)
    out = _tc_epi2(nd4, b2r)
    return out.reshape(-1)[:N]


# speedup vs baseline: 1.2964x; 1.2964x over previous
import functools

import jax
import jax.numpy as jnp
from jax import lax
from jax.experimental import pallas as pl
from jax.experimental.pallas import tpu as pltpu
from jax.experimental.pallas import tpu_sc as plsc

N = 10000
D = 128
HID = 64
HEADS = 8
E = 320000

NPAD = 10240
RPT = NPAD // 16
E_TOT = E + N
E_PAD = 344064
EH = E_PAD // 2
CE1 = E_PAD // 32
K1 = 512
CE2 = E_PAD // 16
K2 = 64
NG = 4
EXTRA = 3 * K2

_mesh = plsc.VectorSubcoreMesh(core_axis_name="c", subcore_axis_name="s")


_R1 = 400


def _tc_prologue_body(x_ref, w_ref, avs_ref, avd_ref, xlg_ref, als_ref, ald_ref):
    xl = jnp.dot(x_ref[...], w_ref[...], preferred_element_type=jnp.float32,
                 precision=lax.Precision.HIGHEST)
    ch = lax.broadcasted_iota(jnp.int32, (HEADS * HID, HEADS), 0)
    hh = lax.broadcasted_iota(jnp.int32, (HEADS * HID, HEADS), 1)
    sel = (ch // HID == hh).astype(jnp.float32)
    als_ref[...] = jnp.dot(xl * avs_ref[...], sel,
                           preferred_element_type=jnp.float32,
                           precision=lax.Precision.HIGHEST)
    ald_ref[...] = jnp.dot(xl * avd_ref[...], sel,
                           preferred_element_type=jnp.float32,
                           precision=lax.Precision.HIGHEST)
    for g in range(NG):
        xlg_ref[g] = xl[:, g * 128:(g + 1) * 128]


def _tc_prologue(x, W1, avs, avd):
    return pl.pallas_call(
        _tc_prologue_body,
        grid=(N // _R1,),
        in_specs=[
            pl.BlockSpec((_R1, D), lambda i: (i, 0)),
            pl.BlockSpec((D, HEADS * HID), lambda i: (0, 0)),
            pl.BlockSpec((1, HEADS * HID), lambda i: (0, 0)),
            pl.BlockSpec((1, HEADS * HID), lambda i: (0, 0)),
        ],
        out_specs=[
            pl.BlockSpec((NG, _R1, 128), lambda i: (0, i, 0)),
            pl.BlockSpec((_R1, HEADS), lambda i: (i, 0)),
            pl.BlockSpec((_R1, HEADS), lambda i: (i, 0)),
        ],
        out_shape=[
            jax.ShapeDtypeStruct((NG, N, 128), jnp.float32),
            jax.ShapeDtypeStruct((N, HEADS), jnp.float32),
            jax.ShapeDtypeStruct((N, HEADS), jnp.float32),
        ],
    )(x, W1, avs, avd)


NCH1 = E_PAD // K1
NI1 = NCH1 // 32
W_LEN = E_PAD * 8 + 2 * K1 * 8


def _sc_pass1_body(idxs_hbm, idxd_hbm, als_hbm, ald_hbm, zeros_hbm,
                   w_hbm, denom_hbm,
                   is0, is1, is2, id0, id1, id2, s0, s1, d0, d1, w0, w1,
                   bounce_v, denom_sh,
                   semL0, semL1, semL2, semG0, semG1, semW0, semW1):
    c = lax.axis_index("c")
    s = lax.axis_index("s")
    wid = c * 16 + s
    idxs = (is0, is1, is2)
    idxd = (id0, id1, id2)
    s_v = (s0, s1)
    d_v = (d0, d1)
    w_v = (w0, w1)
    semL = (semL0, semL1, semL2)
    semG = (semG0, semG1)
    semW = (semW0, semW1)

    z0 = s * (RPT * 8)
    pltpu.sync_copy(zeros_hbm.at[pl.ds(z0, RPT * 8)], bounce_v)
    pltpu.sync_copy(bounce_v, denom_sh.at[pl.ds(z0, RPT * 8)])
    plsc.subcore_barrier()

    def f0_of(i):
        t = jnp.minimum(wid * NI1 + i, NCH1 - 1)
        return t * (K1 * 8)

    def fire_lin(i, l):
        f0 = f0_of(i)
        pltpu.async_copy(idxs_hbm.at[pl.ds(f0, K1 * 8)], idxs[l], semL[l])
        pltpu.async_copy(idxd_hbm.at[pl.ds(f0, K1 * 8)], idxd[l], semL[l])

    def wait_lin(l):
        pltpu.make_async_copy(idxs_hbm.at[pl.ds(0, K1 * 8)], idxs[l],
                              semL[l]).wait()
        pltpu.make_async_copy(idxd_hbm.at[pl.ds(0, K1 * 8)], idxd[l],
                              semL[l]).wait()

    def fire_gather(l, o):
        pltpu.async_copy(als_hbm.at[idxs[l]], s_v[o], semG[o])
        pltpu.async_copy(ald_hbm.at[idxd[l]], d_v[o], semG[o])

    def wait_gather(b):
        pltpu.make_async_copy(als_hbm.at[idxs[0]], s_v[b], semG[b]).wait()
        pltpu.make_async_copy(ald_hbm.at[idxd[0]], d_v[b], semG[b]).wait()

    fire_lin(0, 0)
    fire_lin(1, 1)
    fire_lin(2, 2)
    wait_lin(0)
    fire_gather(0, 0)
    for b in range(2):
        pltpu.async_copy(w_v[b],
                         w_hbm.at[pl.ds(E_PAD * 8 + b * (K1 * 8), K1 * 8)],
                         semW[b])

    def body(i, bi):
        b = bi % 2
        l = bi % 3
        l1 = (bi + 1) % 3
        o = 1 - b
        wait_lin(l1)
        fire_gather(l1, o)
        wait_gather(b)
        pltpu.make_async_copy(w_v[b], w_hbm.at[pl.ds(0, K1 * 8)],
                              semW[b]).wait()

        def vec(j, carry2, b=b):
            e = s_v[b][pl.ds(j * 16, 16)] + d_v[b][pl.ds(j * 16, 16)]
            e = jnp.maximum(e, 0.2 * e)
            w_v[b][pl.ds(j * 16, 16)] = jnp.exp(e)
            return carry2

        lax.fori_loop(0, (K1 * 8) // 16, vec, 0)
        pltpu.async_copy(w_v[b], w_hbm.at[pl.ds(f0_of(i), K1 * 8)],
                         semW[b])
        pltpu.sync_copy(w_v[b], denom_sh.at[idxd[l]], add=True)
        fire_lin(i + 3, l)

    def outer(oi, carry):
        ibase = oi * 6
        for bi in range(6):
            body(ibase + bi, bi)
        return carry

    lax.fori_loop(0, (NI1 // 6) * 6 // 6, outer, 0)
    for j in range(NI1 % 6):
        body((NI1 // 6) * 6 + j, j)
    wait_lin((NI1 + 1) % 3)
    wait_lin((NI1 + 2) % 3)
    wait_gather(NI1 % 2)
    pltpu.make_async_copy(w_v[0], w_hbm.at[pl.ds(0, K1 * 8)], semW[0]).wait()
    pltpu.make_async_copy(w_v[1], w_hbm.at[pl.ds(0, K1 * 8)], semW[1]).wait()

    plsc.subcore_barrier()
    pltpu.sync_copy(denom_sh.at[pl.ds(z0, RPT * 8)], bounce_v)
    pltpu.sync_copy(bounce_v,
                    denom_hbm.at[pl.ds(c * (NPAD * 8) + z0, RPT * 8)])


_sc_pass1 = functools.partial(
    pl.kernel,
    out_type=[
        jax.ShapeDtypeStruct((W_LEN,), jnp.float32),
        jax.ShapeDtypeStruct((2 * NPAD * 8,), jnp.float32),
    ],
    mesh=_mesh,
    scratch_types=[
        pltpu.VMEM((K1 * 8,), jnp.int32),
        pltpu.VMEM((K1 * 8,), jnp.int32),
        pltpu.VMEM((K1 * 8,), jnp.int32),
        pltpu.VMEM((K1 * 8,), jnp.int32),
        pltpu.VMEM((K1 * 8,), jnp.int32),
        pltpu.VMEM((K1 * 8,), jnp.int32),
        pltpu.VMEM((K1 * 8,), jnp.float32),
        pltpu.VMEM((K1 * 8,), jnp.float32),
        pltpu.VMEM((K1 * 8,), jnp.float32),
        pltpu.VMEM((K1 * 8,), jnp.float32),
        pltpu.VMEM((K1 * 8,), jnp.float32),
        pltpu.VMEM((K1 * 8,), jnp.float32),
        pltpu.VMEM((RPT * 8,), jnp.float32),
        pltpu.VMEM_SHARED((NPAD * 8,), jnp.float32),
        pltpu.SemaphoreType.DMA,
        pltpu.SemaphoreType.DMA,
        pltpu.SemaphoreType.DMA,
        pltpu.SemaphoreType.DMA,
        pltpu.SemaphoreType.DMA,
        pltpu.SemaphoreType.DMA,
        pltpu.SemaphoreType.DMA,
    ],
)(_sc_pass1_body)


_QR = RPT // 16


def _sc_pass2_body(xlg_hbm, src_hbm, dst_hbm, w_hbm, zeros_hbm,
                   accum_hbm,
                   src0, src1, src2, dst0, dst1, dst2, idx0, idx1, idx2,
                   w0, w1, w2, rows0, rows1, msg0, msg1, sdst0, sdst1,
                   bounce_v, accum_sh,
                   semL0, semL1, semL2, semG0, semG1, semS0, semS1):
    c = lax.axis_index("c")
    s = lax.axis_index("s")
    r0 = s * RPT
    e_base = s * CE2
    srcs = (src0, src1, src2)
    dsts = (dst0, dst1, dst2)
    idxs = (idx0, idx1, idx2)
    ws = (w0, w1, w2)
    rows = (rows0, rows1)
    msgs = (msg0, msg1)
    sdsts = (sdst0, sdst1)
    semL = (semL0, semL1, semL2)
    semG = (semG0, semG1)
    semS = (semS0, semS1)
    NCH = CE2 // K2

    def fire_lin(cc, l):
        e0 = e_base + cc * K2
        pltpu.async_copy(src_hbm.at[pl.ds(e0, K2)], srcs[l], semL[l])
        pltpu.async_copy(dst_hbm.at[pl.ds(e0, K2)], dsts[l], semL[l])
        pltpu.async_copy(w_hbm.at[pl.ds(e0 * 8, K2 * 8)], ws[l], semL[l])

    def wait_lin(l):
        pltpu.make_async_copy(src_hbm.at[pl.ds(0, K2)], srcs[l],
                              semL[l]).wait()
        pltpu.make_async_copy(dst_hbm.at[pl.ds(0, K2)], dsts[l],
                              semL[l]).wait()
        pltpu.make_async_copy(w_hbm.at[pl.ds(0, K2 * 8)], ws[l],
                              semL[l]).wait()

    def build_idx(l, g):
        for j in range(K2 // 16):
            idxs[l][pl.ds(j * 16, 16)] = (srcs[l][pl.ds(j * 16, 16)]
                                          + g * N)

    for gi in range(2):
        g = 2 * c + gi
        for q in range(16):
            pltpu.sync_copy(zeros_hbm.at[pl.ds(r0 + q * _QR, _QR)], bounce_v)
            pltpu.sync_copy(bounce_v, accum_sh.at[pl.ds(r0 + q * _QR, _QR)])
        plsc.subcore_barrier()

        for b in range(2):
            def zmsg(k, carry, b=b):
                for j in range(8):
                    msgs[b][k, pl.ds(j * 16, 16)] = jnp.zeros((16,),
                                                              jnp.float32)
                return carry
            lax.fori_loop(0, K2, zmsg, 0)
            for j in range(4):
                sdsts[b][pl.ds(j * 16, 16)] = jnp.zeros((16,), jnp.int32)
            pltpu.async_copy(msgs[b], accum_sh.at[sdsts[b]], semS[b],
                             add=True)

        fire_lin(0, 0)
        fire_lin(1, 1)
        fire_lin(2, 2)
        wait_lin(0)
        build_idx(0, g)
        pltpu.async_copy(xlg_hbm.at[idxs[0]], rows[0], semG[0])

        def outer(oi, carry, g=g):
            cbase = oi * 6
            for bi in range(6):
                b = bi % 2
                l = bi % 3
                l1 = (bi + 1) % 3
                o = 1 - b
                cc = cbase + bi
                wait_lin(l1)
                build_idx(l1, g)
                pltpu.async_copy(xlg_hbm.at[idxs[l1]], rows[o], semG[o])
                pltpu.make_async_copy(xlg_hbm.at[idxs[l]], rows[b],
                                      semG[b]).wait()
                pltpu.make_async_copy(msgs[b], accum_sh.at[sdsts[b]],
                                      semS[b]).wait()

                def pair(kk, carry2, b=b, l=l, g=g):
                    wrow = ws[l][pl.ds(kk * 16, 16)]
                    for half in range(2):
                        k = kk * 2 + half
                        lo = half * 8 + 2 * g
                        wlo = wrow.at[jnp.full((16,), lo, jnp.int32)].get(
                            mode="promise_in_bounds")
                        whi = wrow.at[jnp.full((16,), lo + 1,
                                               jnp.int32)].get(
                            mode="promise_in_bounds")
                        for j in range(8):
                            wv = wlo if j < 4 else whi
                            msgs[b][k, pl.ds(j * 16, 16)] = (
                                rows[b][k, pl.ds(j * 16, 16)] * wv)
                    return carry2

                lax.fori_loop(0, K2 // 2, pair, 0)
                for j in range(4):
                    sdsts[b][pl.ds(j * 16, 16)] = dsts[l][pl.ds(j * 16, 16)]
                pltpu.async_copy(msgs[b], accum_sh.at[sdsts[b]], semS[b],
                                 add=True)
                fire_lin(cc + 3, l)
            return carry

        lax.fori_loop(0, NCH // 6, outer, 0)
        wait_lin((NCH + 1) % 3)
        wait_lin((NCH + 2) % 3)
        pltpu.make_async_copy(xlg_hbm.at[idxs[NCH % 3]], rows[NCH % 2],
                              semG[NCH % 2]).wait()
        pltpu.make_async_copy(msgs[0], accum_sh.at[sdsts[0]], semS[0]).wait()
        pltpu.make_async_copy(msgs[1], accum_sh.at[sdsts[1]], semS[1]).wait()
        plsc.subcore_barrier()
        for q in range(16):
            pltpu.sync_copy(accum_sh.at[pl.ds(r0 + q * _QR, _QR)], bounce_v)
            pltpu.sync_copy(bounce_v,
                            accum_hbm.at[g, pl.ds(r0 + q * _QR, _QR)])
        plsc.subcore_barrier()


_sc_pass2 = functools.partial(
    pl.kernel,
    out_type=[jax.ShapeDtypeStruct((NG, NPAD, 128), jnp.float32)],
    mesh=_mesh,
    scratch_types=[
        pltpu.VMEM((K2,), jnp.int32),
        pltpu.VMEM((K2,), jnp.int32),
        pltpu.VMEM((K2,), jnp.int32),
        pltpu.VMEM((K2,), jnp.int32),
        pltpu.VMEM((K2,), jnp.int32),
        pltpu.VMEM((K2,), jnp.int32),
        pltpu.VMEM((K2,), jnp.int32),
        pltpu.VMEM((K2,), jnp.int32),
        pltpu.VMEM((K2,), jnp.int32),
        pltpu.VMEM((K2 * 8,), jnp.float32),
        pltpu.VMEM((K2 * 8,), jnp.float32),
        pltpu.VMEM((K2 * 8,), jnp.float32),
        pltpu.VMEM((K2, 128), jnp.float32),
        pltpu.VMEM((K2, 128), jnp.float32),
        pltpu.VMEM((K2, 128), jnp.float32),
        pltpu.VMEM((K2, 128), jnp.float32),
        pltpu.VMEM((K2,), jnp.int32),
        pltpu.VMEM((K2,), jnp.int32),
        pltpu.VMEM((_QR, 128), jnp.float32),
        pltpu.VMEM_SHARED((NPAD, 128), jnp.float32),
        pltpu.SemaphoreType.DMA,
        pltpu.SemaphoreType.DMA,
        pltpu.SemaphoreType.DMA,
        pltpu.SemaphoreType.DMA,
        pltpu.SemaphoreType.DMA,
        pltpu.SemaphoreType.DMA,
        pltpu.SemaphoreType.DMA,
    ],
)(_sc_pass2_body)


def _tc_epi1_body(acc_ref, dn_ref, b1_ref, w2_ref, z_ref):
    dn = dn_ref[0] + dn_ref[1] + 1e-16
    rdn = 1.0 / dn
    ch = lax.broadcasted_iota(jnp.int32, (HEADS, HEADS * HID), 1)
    hh = lax.broadcasted_iota(jnp.int32, (HEADS, HEADS * HID), 0)
    sel = (ch // HID == hh).astype(jnp.float32)
    rdn_full = jnp.dot(rdn, sel, preferred_element_type=jnp.float32,
                       precision=lax.Precision.HIGHEST)
    acc = jnp.concatenate([acc_ref[g] for g in range(NG)], axis=1)
    h = jnp.maximum(acc * rdn_full + b1_ref[...], 0.0)
    z_ref[...] = jnp.dot(h, w2_ref[...], preferred_element_type=jnp.float32,
                         precision=lax.Precision.HIGHEST)


_RE1 = 1280


def _tc_epi1(accum, denom, b1, W2):
    return pl.pallas_call(
        _tc_epi1_body,
        grid=(NPAD // _RE1,),
        in_specs=[
            pl.BlockSpec((NG, _RE1, 128), lambda i: (0, i, 0)),
            pl.BlockSpec((2, _RE1, HEADS), lambda i: (0, i, 0)),
            pl.BlockSpec((1, HEADS * HID), lambda i: (0, 0)),
            pl.BlockSpec((HEADS * HID, 1), lambda i: (0, 0)),
        ],
        out_specs=pl.BlockSpec((_RE1, 1), lambda i: (i, 0)),
        out_shape=jax.ShapeDtypeStruct((NPAD, 1), jnp.float32),
    )(accum, denom, b1, W2)


def _sc_pass3_body(src_hbm, dst_hbm, als_hbm, ald_hbm, z_hbm, zeros_hbm,
                   nd_hbm,
                   src_v, dst_v, s_v, d_v, z_v, w_v, m_v, bounce_v, nd_sh,
                   sem1, sem2, sem3):
    c = lax.axis_index("c")
    s = lax.axis_index("s")
    z0 = s * (2 * RPT)
    pltpu.sync_copy(zeros_hbm.at[pl.ds(z0, 2 * RPT)], bounce_v)
    pltpu.sync_copy(bounce_v, nd_sh.at[pl.ds(z0, 2 * RPT)])
    plsc.subcore_barrier()

    e_base = c * EH + s * CE1

    def chunk(i, carry):
        e0 = e_base + i * K1
        pltpu.sync_copy(src_hbm.at[pl.ds(e0, K1)], src_v)
        pltpu.sync_copy(dst_hbm.at[pl.ds(e0, K1)], dst_v)
        pltpu.async_copy(als_hbm.at[src_v], s_v, sem1).wait()
        pltpu.async_copy(ald_hbm.at[dst_v], d_v, sem2).wait()
        pltpu.async_copy(z_hbm.at[src_v], z_v, sem3).wait()

        def vec(j, carry2):
            e = s_v[pl.ds(j * 16, 16)] + d_v[pl.ds(j * 16, 16)]
            e = jnp.maximum(e, 0.2 * e)
            w = jnp.exp(e)
            w_v[pl.ds(j * 16, 16)] = w
            m_v[pl.ds(j * 16, 16)] = w * z_v[pl.ds(j * 16, 16)]
            return carry2

        lax.fori_loop(0, K1 // 16, vec, 0)
        pltpu.sync_copy(w_v, nd_sh.at[dst_v], add=True)

        def mkidx(j, carry2):
            dst_v[pl.ds(j * 16, 16)] = dst_v[pl.ds(j * 16, 16)] + NPAD
            return carry2

        lax.fori_loop(0, K1 // 16, mkidx, 0)
        pltpu.sync_copy(m_v, nd_sh.at[dst_v], add=True)
        return carry

    lax.fori_loop(0, CE1 // K1, chunk, 0)
    plsc.subcore_barrier()
    pltpu.sync_copy(nd_sh.at[pl.ds(z0, 2 * RPT)], bounce_v)
    pltpu.sync_copy(bounce_v,
                    nd_hbm.at[pl.ds(c * (2 * NPAD) + z0, 2 * RPT)])


_sc_pass3 = functools.partial(
    pl.kernel,
    out_type=[jax.ShapeDtypeStruct((2 * 2 * NPAD,), jnp.float32)],
    mesh=_mesh,
    scratch_types=[
        pltpu.VMEM((K1,), jnp.int32),
        pltpu.VMEM((K1,), jnp.int32),
        pltpu.VMEM((K1,), jnp.float32),
        pltpu.VMEM((K1,), jnp.float32),
        pltpu.VMEM((K1,), jnp.float32),
        pltpu.VMEM((K1,), jnp.float32),
        pltpu.VMEM((K1,), jnp.float32),
        pltpu.VMEM((2 * RPT,), jnp.float32),
        pltpu.VMEM_SHARED((2 * NPAD,), jnp.float32),
        pltpu.SemaphoreType.DMA,
        pltpu.SemaphoreType.DMA,
        pltpu.SemaphoreType.DMA,
    ],
)(_sc_pass3_body)


def _tc_epi2_body(nd_ref, b2_ref, o_ref):
    dn = nd_ref[0, 0] + nd_ref[1, 0]
    nm = nd_ref[0, 1] + nd_ref[1, 1]
    o_ref[...] = nm / (dn + 1e-16) + b2_ref[0, 0]


def _tc_epi2(nd, b2):
    return pl.pallas_call(
        _tc_epi2_body,
        in_specs=[
            pl.BlockSpec((2, 2, NPAD // 64, 64), lambda: (0, 0, 0, 0)),
            pl.BlockSpec(memory_space=pltpu.SMEM),
        ],
        out_specs=pl.BlockSpec((NPAD // 64, 64), lambda: (0, 0)),
        out_shape=jax.ShapeDtypeStruct((NPAD // 64, 64), jnp.float32),
        grid=(),
    )(nd, b2)


def kernel(x, edge_index, W1, a_s1, a_d1, b1, W2, a_s2, a_d2, b2):
    f32 = jnp.float32
    loop = jnp.arange(N, dtype=jnp.int32)
    src = jnp.concatenate([edge_index[0].astype(jnp.int32), loop])
    dst = jnp.concatenate([edge_index[1].astype(jnp.int32), loop])
    npad_e = E_PAD - E_TOT
    pidx = jnp.arange(npad_e, dtype=jnp.int32)
    ztail = jnp.zeros((EXTRA,), jnp.int32)
    src_full = jnp.concatenate([src, pidx % N, ztail])
    dst_full = jnp.concatenate([dst, N + (pidx % 16), ztail])

    h8 = jnp.arange(8, dtype=jnp.int32)
    idx_s8 = (src_full[:E_PAD, None] * 8 + h8[None, :]).reshape(-1)
    idx_d8 = (dst_full[:E_PAD, None] * 8 + h8[None, :]).reshape(-1)

    avs = a_s1.reshape(1, HEADS * HID)
    avd = a_d1.reshape(1, HEADS * HID)
    xlg, als, ald = _tc_prologue(x, W1, avs, avd)
    xlg_flat = xlg.reshape(NG * N, 128)
    als_flat = jnp.pad(als, ((0, NPAD - N), (0, 0))).reshape(-1)
    ald_flat = jnp.pad(ald, ((0, NPAD - N), (0, 0))).reshape(-1)

    zeros8 = jnp.zeros((NPAD * 8,), f32)
    w_flat, denom = _sc_pass1(idx_s8, idx_d8, als_flat, ald_flat, zeros8)

    zeros128 = jnp.zeros((NPAD, 128), f32)
    (accum,) = _sc_pass2(xlg_flat, src_full, dst_full, w_flat, zeros128)

    denom3 = denom.reshape(2, NPAD, HEADS)
    b1r = b1.reshape(1, HEADS * HID)
    z = _tc_epi1(accum, denom3, b1r, W2)

    z_flat = z.reshape(-1)
    as2 = a_s2.reshape(())
    ad2 = a_d2.reshape(())
    als2 = z_flat * as2
    ald2 = z_flat * ad2

    zeros2 = jnp.zeros((2 * NPAD,), f32)
    (nd,) = _sc_pass3(src_full, dst_full, als2, ald2, z_flat, zeros2)

    nd4 = nd.reshape(2, 2, NPAD // 64, 64)
    b2r = b2.reshape(1, 1)
    out = _tc_epi2(nd4, b2r)
    return out.reshape(-1)[:N]


# speedup vs baseline: 1.5954x; 1.2307x over previous
import functools

import jax
import jax.numpy as jnp
from jax import lax
from jax.experimental import pallas as pl
from jax.experimental.pallas import tpu as pltpu
from jax.experimental.pallas import tpu_sc as plsc

N = 10000
D = 128
HID = 64
HEADS = 8
E = 320000

NPAD = 10240
RPT = NPAD // 16
E_TOT = E + N
E_PAD = 344064
EH = E_PAD // 2
CE1 = E_PAD // 32
K1 = 512
CE2 = E_PAD // 16
K2 = 64
NG = 4
EXTRA = 3 * K2

_mesh = plsc.VectorSubcoreMesh(core_axis_name="c", subcore_axis_name="s")


_R1 = 400


def _tc_prologue_body(x_ref, w_ref, avs_ref, avd_ref, xlg_ref, als_ref, ald_ref):
    xl = jnp.dot(x_ref[...], w_ref[...], preferred_element_type=jnp.float32,
                 precision=lax.Precision.HIGHEST)
    ch = lax.broadcasted_iota(jnp.int32, (HEADS * HID, HEADS), 0)
    hh = lax.broadcasted_iota(jnp.int32, (HEADS * HID, HEADS), 1)
    sel = (ch // HID == hh).astype(jnp.float32)
    als_ref[...] = jnp.dot(xl * avs_ref[...], sel,
                           preferred_element_type=jnp.float32,
                           precision=lax.Precision.HIGHEST)
    ald_ref[...] = jnp.dot(xl * avd_ref[...], sel,
                           preferred_element_type=jnp.float32,
                           precision=lax.Precision.HIGHEST)
    for g in range(NG):
        xlg_ref[g] = xl[:, g * 128:(g + 1) * 128]


def _tc_prologue(x, W1, avs, avd):
    return pl.pallas_call(
        _tc_prologue_body,
        grid=(N // _R1,),
        in_specs=[
            pl.BlockSpec((_R1, D), lambda i: (i, 0)),
            pl.BlockSpec((D, HEADS * HID), lambda i: (0, 0)),
            pl.BlockSpec((1, HEADS * HID), lambda i: (0, 0)),
            pl.BlockSpec((1, HEADS * HID), lambda i: (0, 0)),
        ],
        out_specs=[
            pl.BlockSpec((NG, _R1, 128), lambda i: (0, i, 0)),
            pl.BlockSpec((_R1, HEADS), lambda i: (i, 0)),
            pl.BlockSpec((_R1, HEADS), lambda i: (i, 0)),
        ],
        out_shape=[
            jax.ShapeDtypeStruct((NG, N, 128), jnp.float32),
            jax.ShapeDtypeStruct((N, HEADS), jnp.float32),
            jax.ShapeDtypeStruct((N, HEADS), jnp.float32),
        ],
    )(x, W1, avs, avd)


NCH1 = E_PAD // K1
NI1 = NCH1 // 32
W_LEN = E_PAD * 8 + 2 * K1 * 8


def _sc_pass1_body(src_hbm, dst_hbm, als_hbm, ald_hbm, zeros_hbm,
                   w_hbm, denom_hbm,
                   sr0, sr1, sr2, dr0, dr1, dr2, is0, is1, id0, id1,
                   s0, s1, d0, d1, w0, w1,
                   bounce_v, denom_sh,
                   semL0, semL1, semL2, semG0, semG1, semW0, semW1):
    c = lax.axis_index("c")
    s = lax.axis_index("s")
    wid = c * 16 + s
    srcs = (sr0, sr1, sr2)
    dsts = (dr0, dr1, dr2)
    idxs = (is0, is1)
    idxd = (id0, id1)
    s_v = (s0, s1)
    d_v = (d0, d1)
    w_v = (w0, w1)
    semL = (semL0, semL1, semL2)
    semG = (semG0, semG1)
    semW = (semW0, semW1)

    z0 = s * (RPT * 8)
    pltpu.sync_copy(zeros_hbm.at[pl.ds(z0, RPT * 8)], bounce_v)
    pltpu.sync_copy(bounce_v, denom_sh.at[pl.ds(z0, RPT * 8)])
    plsc.subcore_barrier()

    def f0_of(i):
        t = jnp.minimum(wid * NI1 + i, NCH1 - 1)
        return t * (K1 * 8)

    def e0_of(i):
        t = jnp.minimum(wid * NI1 + i, NCH1 - 1)
        return t * K1

    def fire_lin(i, l):
        e0 = e0_of(i)
        pltpu.async_copy(src_hbm.at[pl.ds(e0, K1)], srcs[l], semL[l])
        pltpu.async_copy(dst_hbm.at[pl.ds(e0, K1)], dsts[l], semL[l])

    def wait_lin(l):
        pltpu.make_async_copy(src_hbm.at[pl.ds(0, K1)], srcs[l],
                              semL[l]).wait()
        pltpu.make_async_copy(dst_hbm.at[pl.ds(0, K1)], dsts[l],
                              semL[l]).wait()

    def build_idx(l, o):
        def bidx(j, carry2):
            win_s = srcs[l][pl.ds(j * 16, 16)] * 8
            win_d = dsts[l][pl.ds(j * 16, 16)] * 8
            for h in range(8):
                idxs[o][pl.ds(h * K1 + j * 16, 16)] = win_s + h
                idxd[o][pl.ds(h * K1 + j * 16, 16)] = win_d + h
            return carry2

        lax.fori_loop(0, K1 // 16, bidx, 0)

    def fire_gather(o):
        pltpu.async_copy(als_hbm.at[idxs[o]], s_v[o], semG[o])
        pltpu.async_copy(ald_hbm.at[idxd[o]], d_v[o], semG[o])

    def wait_gather(b):
        pltpu.make_async_copy(als_hbm.at[idxs[0]], s_v[b], semG[b]).wait()
        pltpu.make_async_copy(ald_hbm.at[idxd[0]], d_v[b], semG[b]).wait()

    fire_lin(0, 0)
    fire_lin(1, 1)
    fire_lin(2, 2)
    wait_lin(0)
    build_idx(0, 0)
    fire_gather(0)
    for b in range(2):
        pltpu.async_copy(w_v[b],
                         w_hbm.at[pl.ds(E_PAD * 8 + b * (K1 * 8), K1 * 8)],
                         semW[b])

    def body(i, bi):
        b = bi % 2
        l = bi % 3
        l1 = (bi + 1) % 3
        o = 1 - b
        wait_lin(l1)
        build_idx(l1, o)
        fire_gather(o)
        wait_gather(b)
        pltpu.make_async_copy(w_v[b], w_hbm.at[pl.ds(0, K1 * 8)],
                              semW[b]).wait()

        def vec(j, carry2, b=b):
            e = s_v[b][pl.ds(j * 16, 16)] + d_v[b][pl.ds(j * 16, 16)]
            e = jnp.maximum(e, 0.2 * e)
            w_v[b][pl.ds(j * 16, 16)] = jnp.exp(e)
            return carry2

        lax.fori_loop(0, (K1 * 8) // 16, vec, 0)
        pltpu.async_copy(w_v[b], w_hbm.at[pl.ds(f0_of(i), K1 * 8)],
                         semW[b])
        pltpu.sync_copy(w_v[b], denom_sh.at[idxd[b]], add=True)
        fire_lin(i + 3, l)

    def outer(oi, carry):
        ibase = oi * 6
        for bi in range(6):
            body(ibase + bi, bi)
        return carry

    lax.fori_loop(0, (NI1 // 6) * 6 // 6, outer, 0)
    for j in range(NI1 % 6):
        body((NI1 // 6) * 6 + j, j)
    wait_lin((NI1 + 1) % 3)
    wait_lin((NI1 + 2) % 3)
    wait_gather(NI1 % 2)
    pltpu.make_async_copy(w_v[0], w_hbm.at[pl.ds(0, K1 * 8)], semW[0]).wait()
    pltpu.make_async_copy(w_v[1], w_hbm.at[pl.ds(0, K1 * 8)], semW[1]).wait()

    plsc.subcore_barrier()
    pltpu.sync_copy(denom_sh.at[pl.ds(z0, RPT * 8)], bounce_v)
    pltpu.sync_copy(bounce_v,
                    denom_hbm.at[pl.ds(c * (NPAD * 8) + z0, RPT * 8)])


_sc_pass1 = functools.partial(
    pl.kernel,
    out_type=[
        jax.ShapeDtypeStruct((W_LEN,), jnp.float32),
        jax.ShapeDtypeStruct((2 * NPAD * 8,), jnp.float32),
    ],
    mesh=_mesh,
    scratch_types=[
        pltpu.VMEM((K1,), jnp.int32),
        pltpu.VMEM((K1,), jnp.int32),
        pltpu.VMEM((K1,), jnp.int32),
        pltpu.VMEM((K1,), jnp.int32),
        pltpu.VMEM((K1,), jnp.int32),
        pltpu.VMEM((K1,), jnp.int32),
        pltpu.VMEM((K1 * 8,), jnp.int32),
        pltpu.VMEM((K1 * 8,), jnp.int32),
        pltpu.VMEM((K1 * 8,), jnp.int32),
        pltpu.VMEM((K1 * 8,), jnp.int32),
        pltpu.VMEM((K1 * 8,), jnp.float32),
        pltpu.VMEM((K1 * 8,), jnp.float32),
        pltpu.VMEM((K1 * 8,), jnp.float32),
        pltpu.VMEM((K1 * 8,), jnp.float32),
        pltpu.VMEM((K1 * 8,), jnp.float32),
        pltpu.VMEM((K1 * 8,), jnp.float32),
        pltpu.VMEM((RPT * 8,), jnp.float32),
        pltpu.VMEM_SHARED((NPAD * 8,), jnp.float32),
        pltpu.SemaphoreType.DMA,
        pltpu.SemaphoreType.DMA,
        pltpu.SemaphoreType.DMA,
        pltpu.SemaphoreType.DMA,
        pltpu.SemaphoreType.DMA,
        pltpu.SemaphoreType.DMA,
        pltpu.SemaphoreType.DMA,
    ],
)(_sc_pass1_body)


_QR = RPT // 16


def _sc_pass2_body(xlg_hbm, src_hbm, dst_hbm, w_hbm, zeros_hbm,
                   accum_hbm,
                   src0, src1, src2, dst0, dst1, dst2, idx0, idx1, idx2,
                   wl0, wl1, wl2, wh0, wh1, wh2, rows0, rows1, msg0, msg1,
                   sdst0, sdst1,
                   bounce_v, accum_sh,
                   semL0, semL1, semL2, semG0, semG1, semS0, semS1):
    c = lax.axis_index("c")
    s = lax.axis_index("s")
    r0 = s * RPT
    e_base = s * CE2
    srcs = (src0, src1, src2)
    dsts = (dst0, dst1, dst2)
    idxs = (idx0, idx1, idx2)
    wlo = (wl0, wl1, wl2)
    whi = (wh0, wh1, wh2)
    rows = (rows0, rows1)
    msgs = (msg0, msg1)
    sdsts = (sdst0, sdst1)
    semL = (semL0, semL1, semL2)
    semG = (semG0, semG1)
    semS = (semS0, semS1)
    NCH = CE2 // K2

    def fire_lin(cc, l, g):
        e0 = e_base + cc * K2
        t = e0 // K1
        off = t * (K1 * 8) + (2 * g) * K1 + (e0 - t * K1)
        pltpu.async_copy(src_hbm.at[pl.ds(e0, K2)], srcs[l], semL[l])
        pltpu.async_copy(dst_hbm.at[pl.ds(e0, K2)], dsts[l], semL[l])
        pltpu.async_copy(w_hbm.at[pl.ds(off, K2)], wlo[l], semL[l])
        pltpu.async_copy(w_hbm.at[pl.ds(off + K1, K2)], whi[l], semL[l])

    def wait_lin(l):
        pltpu.make_async_copy(src_hbm.at[pl.ds(0, K2)], srcs[l],
                              semL[l]).wait()
        pltpu.make_async_copy(dst_hbm.at[pl.ds(0, K2)], dsts[l],
                              semL[l]).wait()
        pltpu.make_async_copy(w_hbm.at[pl.ds(0, K2)], wlo[l],
                              semL[l]).wait()
        pltpu.make_async_copy(w_hbm.at[pl.ds(0, K2)], whi[l],
                              semL[l]).wait()

    def build_idx(l, g):
        for j in range(K2 // 16):
            idxs[l][pl.ds(j * 16, 16)] = (srcs[l][pl.ds(j * 16, 16)]
                                          + g * N)

    for gi in range(2):
        g = 2 * c + gi
        for q in range(16):
            pltpu.sync_copy(zeros_hbm.at[pl.ds(r0 + q * _QR, _QR)], bounce_v)
            pltpu.sync_copy(bounce_v, accum_sh.at[pl.ds(r0 + q * _QR, _QR)])
        plsc.subcore_barrier()

        for b in range(2):
            def zmsg(k, carry, b=b):
                for j in range(8):
                    msgs[b][k, pl.ds(j * 16, 16)] = jnp.zeros((16,),
                                                              jnp.float32)
                return carry
            lax.fori_loop(0, K2, zmsg, 0)
            for j in range(4):
                sdsts[b][pl.ds(j * 16, 16)] = jnp.zeros((16,), jnp.int32)
            pltpu.async_copy(msgs[b], accum_sh.at[sdsts[b]], semS[b],
                             add=True)

        fire_lin(0, 0, g)
        fire_lin(1, 1, g)
        fire_lin(2, 2, g)
        wait_lin(0)
        build_idx(0, g)
        pltpu.async_copy(xlg_hbm.at[idxs[0]], rows[0], semG[0])

        def outer(oi, carry, g=g):
            cbase = oi * 6
            for bi in range(6):
                b = bi % 2
                l = bi % 3
                l1 = (bi + 1) % 3
                o = 1 - b
                cc = cbase + bi
                wait_lin(l1)
                build_idx(l1, g)
                pltpu.async_copy(xlg_hbm.at[idxs[l1]], rows[o], semG[o])
                pltpu.make_async_copy(xlg_hbm.at[idxs[l]], rows[b],
                                      semG[b]).wait()
                pltpu.make_async_copy(msgs[b], accum_sh.at[sdsts[b]],
                                      semS[b]).wait()

                def block(bb, carry2, b=b, l=l):
                    win_lo = wlo[l][pl.ds(bb * 16, 16)]
                    win_hi = whi[l][pl.ds(bb * 16, 16)]
                    for ee in range(16):
                        k = bb * 16 + ee
                        sel = jnp.full((16,), ee, jnp.int32)
                        vlo = win_lo.at[sel].get(mode="promise_in_bounds")
                        vhi = win_hi.at[sel].get(mode="promise_in_bounds")
                        for j in range(8):
                            wv = vlo if j < 4 else vhi
                            msgs[b][k, pl.ds(j * 16, 16)] = (
                                rows[b][k, pl.ds(j * 16, 16)] * wv)
                    return carry2

                lax.fori_loop(0, K2 // 16, block, 0)
                for j in range(4):
                    sdsts[b][pl.ds(j * 16, 16)] = dsts[l][pl.ds(j * 16, 16)]
                pltpu.async_copy(msgs[b], accum_sh.at[sdsts[b]], semS[b],
                                 add=True)
                fire_lin(cc + 3, l, g)
            return carry

        lax.fori_loop(0, NCH // 6, outer, 0)
        wait_lin((NCH + 1) % 3)
        wait_lin((NCH + 2) % 3)
        pltpu.make_async_copy(xlg_hbm.at[idxs[NCH % 3]], rows[NCH % 2],
                              semG[NCH % 2]).wait()
        pltpu.make_async_copy(msgs[0], accum_sh.at[sdsts[0]], semS[0]).wait()
        pltpu.make_async_copy(msgs[1], accum_sh.at[sdsts[1]], semS[1]).wait()
        plsc.subcore_barrier()
        for q in range(16):
            pltpu.sync_copy(accum_sh.at[pl.ds(r0 + q * _QR, _QR)], bounce_v)
            pltpu.sync_copy(bounce_v,
                            accum_hbm.at[g, pl.ds(r0 + q * _QR, _QR)])
        plsc.subcore_barrier()


_sc_pass2 = functools.partial(
    pl.kernel,
    out_type=[jax.ShapeDtypeStruct((NG, NPAD, 128), jnp.float32)],
    mesh=_mesh,
    scratch_types=[
        pltpu.VMEM((K2,), jnp.int32),
        pltpu.VMEM((K2,), jnp.int32),
        pltpu.VMEM((K2,), jnp.int32),
        pltpu.VMEM((K2,), jnp.int32),
        pltpu.VMEM((K2,), jnp.int32),
        pltpu.VMEM((K2,), jnp.int32),
        pltpu.VMEM((K2,), jnp.int32),
        pltpu.VMEM((K2,), jnp.int32),
        pltpu.VMEM((K2,), jnp.int32),
        pltpu.VMEM((K2,), jnp.float32),
        pltpu.VMEM((K2,), jnp.float32),
        pltpu.VMEM((K2,), jnp.float32),
        pltpu.VMEM((K2,), jnp.float32),
        pltpu.VMEM((K2,), jnp.float32),
        pltpu.VMEM((K2,), jnp.float32),
        pltpu.VMEM((K2, 128), jnp.float32),
        pltpu.VMEM((K2, 128), jnp.float32),
        pltpu.VMEM((K2, 128), jnp.float32),
        pltpu.VMEM((K2, 128), jnp.float32),
        pltpu.VMEM((K2,), jnp.int32),
        pltpu.VMEM((K2,), jnp.int32),
        pltpu.VMEM((_QR, 128), jnp.float32),
        pltpu.VMEM_SHARED((NPAD, 128), jnp.float32),
        pltpu.SemaphoreType.DMA,
        pltpu.SemaphoreType.DMA,
        pltpu.SemaphoreType.DMA,
        pltpu.SemaphoreType.DMA,
        pltpu.SemaphoreType.DMA,
        pltpu.SemaphoreType.DMA,
        pltpu.SemaphoreType.DMA,
    ],
)(_sc_pass2_body)


def _tc_epi1_body(acc_ref, dn_ref, b1_ref, w2_ref, z_ref):
    dn = dn_ref[0] + dn_ref[1] + 1e-16
    rdn = 1.0 / dn
    ch = lax.broadcasted_iota(jnp.int32, (HEADS, HEADS * HID), 1)
    hh = lax.broadcasted_iota(jnp.int32, (HEADS, HEADS * HID), 0)
    sel = (ch // HID == hh).astype(jnp.float32)
    rdn_full = jnp.dot(rdn, sel, preferred_element_type=jnp.float32,
                       precision=lax.Precision.HIGHEST)
    acc = jnp.concatenate([acc_ref[g] for g in range(NG)], axis=1)
    h = jnp.maximum(acc * rdn_full + b1_ref[...], 0.0)
    z_ref[...] = jnp.dot(h, w2_ref[...], preferred_element_type=jnp.float32,
                         precision=lax.Precision.HIGHEST)


_RE1 = 1280


def _tc_epi1(accum, denom, b1, W2):
    return pl.pallas_call(
        _tc_epi1_body,
        grid=(NPAD // _RE1,),
        in_specs=[
            pl.BlockSpec((NG, _RE1, 128), lambda i: (0, i, 0)),
            pl.BlockSpec((2, _RE1, HEADS), lambda i: (0, i, 0)),
            pl.BlockSpec((1, HEADS * HID), lambda i: (0, 0)),
            pl.BlockSpec((HEADS * HID, 1), lambda i: (0, 0)),
        ],
        out_specs=pl.BlockSpec((_RE1, 1), lambda i: (i, 0)),
        out_shape=jax.ShapeDtypeStruct((NPAD, 1), jnp.float32),
    )(accum, denom, b1, W2)


def _sc_pass3_body(src_hbm, dst_hbm, als_hbm, ald_hbm, z_hbm, zeros_hbm,
                   nd_hbm,
                   src_v, dst_v, s_v, d_v, z_v, w_v, m_v, bounce_v, nd_sh,
                   sem1, sem2, sem3):
    c = lax.axis_index("c")
    s = lax.axis_index("s")
    z0 = s * (2 * RPT)
    pltpu.sync_copy(zeros_hbm.at[pl.ds(z0, 2 * RPT)], bounce_v)
    pltpu.sync_copy(bounce_v, nd_sh.at[pl.ds(z0, 2 * RPT)])
    plsc.subcore_barrier()

    e_base = c * EH + s * CE1

    def chunk(i, carry):
        e0 = e_base + i * K1
        pltpu.sync_copy(src_hbm.at[pl.ds(e0, K1)], src_v)
        pltpu.sync_copy(dst_hbm.at[pl.ds(e0, K1)], dst_v)
        pltpu.async_copy(als_hbm.at[src_v], s_v, sem1).wait()
        pltpu.async_copy(ald_hbm.at[dst_v], d_v, sem2).wait()
        pltpu.async_copy(z_hbm.at[src_v], z_v, sem3).wait()

        def vec(j, carry2):
            e = s_v[pl.ds(j * 16, 16)] + d_v[pl.ds(j * 16, 16)]
            e = jnp.maximum(e, 0.2 * e)
            w = jnp.exp(e)
            w_v[pl.ds(j * 16, 16)] = w
            m_v[pl.ds(j * 16, 16)] = w * z_v[pl.ds(j * 16, 16)]
            return carry2

        lax.fori_loop(0, K1 // 16, vec, 0)
        pltpu.sync_copy(w_v, nd_sh.at[dst_v], add=True)

        def mkidx(j, carry2):
            dst_v[pl.ds(j * 16, 16)] = dst_v[pl.ds(j * 16, 16)] + NPAD
            return carry2

        lax.fori_loop(0, K1 // 16, mkidx, 0)
        pltpu.sync_copy(m_v, nd_sh.at[dst_v], add=True)
        return carry

    lax.fori_loop(0, CE1 // K1, chunk, 0)
    plsc.subcore_barrier()
    pltpu.sync_copy(nd_sh.at[pl.ds(z0, 2 * RPT)], bounce_v)
    pltpu.sync_copy(bounce_v,
                    nd_hbm.at[pl.ds(c * (2 * NPAD) + z0, 2 * RPT)])


_sc_pass3 = functools.partial(
    pl.kernel,
    out_type=[jax.ShapeDtypeStruct((2 * 2 * NPAD,), jnp.float32)],
    mesh=_mesh,
    scratch_types=[
        pltpu.VMEM((K1,), jnp.int32),
        pltpu.VMEM((K1,), jnp.int32),
        pltpu.VMEM((K1,), jnp.float32),
        pltpu.VMEM((K1,), jnp.float32),
        pltpu.VMEM((K1,), jnp.float32),
        pltpu.VMEM((K1,), jnp.float32),
        pltpu.VMEM((K1,), jnp.float32),
        pltpu.VMEM((2 * RPT,), jnp.float32),
        pltpu.VMEM_SHARED((2 * NPAD,), jnp.float32),
        pltpu.SemaphoreType.DMA,
        pltpu.SemaphoreType.DMA,
        pltpu.SemaphoreType.DMA,
    ],
)(_sc_pass3_body)


def _tc_epi2_body(nd_ref, b2_ref, o_ref):
    dn = nd_ref[0, 0] + nd_ref[1, 0]
    nm = nd_ref[0, 1] + nd_ref[1, 1]
    o_ref[...] = nm / (dn + 1e-16) + b2_ref[0, 0]


def _tc_epi2(nd, b2):
    return pl.pallas_call(
        _tc_epi2_body,
        in_specs=[
            pl.BlockSpec((2, 2, NPAD // 64, 64), lambda: (0, 0, 0, 0)),
            pl.BlockSpec(memory_space=pltpu.SMEM),
        ],
        out_specs=pl.BlockSpec((NPAD // 64, 64), lambda: (0, 0)),
        out_shape=jax.ShapeDtypeStruct((NPAD // 64, 64), jnp.float32),
        grid=(),
    )(nd, b2)


def kernel(x, edge_index, W1, a_s1, a_d1, b1, W2, a_s2, a_d2, b2):
    f32 = jnp.float32
    loop = jnp.arange(N, dtype=jnp.int32)
    src = jnp.concatenate([edge_index[0].astype(jnp.int32), loop])
    dst = jnp.concatenate([edge_index[1].astype(jnp.int32), loop])
    npad_e = E_PAD - E_TOT
    pidx = jnp.arange(npad_e, dtype=jnp.int32)
    ztail = jnp.zeros((EXTRA,), jnp.int32)
    src_full = jnp.concatenate([src, pidx % N, ztail])
    dst_full = jnp.concatenate([dst, N + (pidx % 16), ztail])

    avs = a_s1.reshape(1, HEADS * HID)
    avd = a_d1.reshape(1, HEADS * HID)
    xlg, als, ald = _tc_prologue(x, W1, avs, avd)
    xlg_flat = xlg.reshape(NG * N, 128)
    als_flat = jnp.pad(als, ((0, NPAD - N), (0, 0))).reshape(-1)
    ald_flat = jnp.pad(ald, ((0, NPAD - N), (0, 0))).reshape(-1)

    zeros8 = jnp.zeros((NPAD * 8,), f32)
    w_flat, denom = _sc_pass1(src_full, dst_full, als_flat, ald_flat,
                              zeros8)

    zeros128 = jnp.zeros((NPAD, 128), f32)
    (accum,) = _sc_pass2(xlg_flat, src_full, dst_full, w_flat, zeros128)

    denom3 = denom.reshape(2, NPAD, HEADS)
    b1r = b1.reshape(1, HEADS * HID)
    z = _tc_epi1(accum, denom3, b1r, W2)

    z_flat = z.reshape(-1)
    as2 = a_s2.reshape(())
    ad2 = a_d2.reshape(())
    als2 = z_flat * as2
    ald2 = z_flat * ad2

    zeros2 = jnp.zeros((2 * NPAD,), f32)
    (nd,) = _sc_pass3(src_full, dst_full, als2, ald2, z_flat, zeros2)

    nd4 = nd.reshape(2, 2, NPAD // 64, 64)
    b2r = b2.reshape(1, 1)
    out = _tc_epi2(nd4, b2r)
    return out.reshape(-1)[:N]
